# Initial kernel scaffold; baseline (speedup 1.0000x reference)
import jax
import jax.numpy as jnp
from jax.experimental import pallas as pl


def kernel(word_prob, prob, stops, word_length):
    raise NotImplementedError("write your pallas kernel here")


# speedup vs baseline: 1.3226x; 1.3226x over previous
import jax
import jax.numpy as jnp
from jax.experimental import pallas as pl

_BEAM = 8
_PAD = 0
_EOS = 2
_NEG = -3.0e38
_IMAX = 2147483647


def _group_kernel(wp_ref, p_ref, st_ref, wl_ref, bs_ref, nw_ref, pi_ref, np_ref):
    g = pl.program_id(0)
    V = wp_ref.shape[1]
    w = wp_ref[...]
    p = p_ref[0, 0, :].reshape(_BEAM, 1)
    st = st_ref[0, 0, :].reshape(_BEAM, 1)
    wl = wl_ref[0, 0, :].reshape(_BEAM, 1)
    stf = st.astype(jnp.float32)

    col = jax.lax.broadcasted_iota(jnp.int32, (_BEAM, V), 1)
    row = jax.lax.broadcasted_iota(jnp.int32, (_BEAM, V), 0)

    pad_col = (col == _PAD).astype(jnp.float32)
    wp = p * (w * (1.0 - stf) + pad_col * stf)
    unstop = jnp.where((col == _PAD) | (col == _EOS), 0, 1)
    wl_full = wl + unstop * (1 - st)
    lp = (wl_full + 5).astype(jnp.float32) * (1.0 / 6.0)
    s = jnp.log(jnp.clip(wp, 1e-20, 1.0)) / lp

    flat = row * V + col
    lane = jax.lax.broadcasted_iota(jnp.int32, (1, 1, _BEAM), 2)
    bs_v = jnp.zeros((1, 1, _BEAM), jnp.float32)
    nw_v = jnp.zeros((1, 1, _BEAM), jnp.int32)
    pi_v = jnp.zeros((1, 1, _BEAM), jnp.int32)
    np_v = jnp.zeros((1, 1, _BEAM), jnp.float32)
    for k in range(_BEAM):
        m = jnp.max(s)
        sel = jnp.min(jnp.where(s == m, flat, _IMAX))
        wsel = jnp.max(jnp.where(flat == sel, wp, -1.0))
        bs_v = jnp.where(lane == k, m, bs_v)
        nw_v = jnp.where(lane == k, sel % V, nw_v)
        pi_v = jnp.where(lane == k, g * _BEAM + sel // V, pi_v)
        np_v = jnp.where(lane == k, wsel, np_v)
        if k < _BEAM - 1:
            s = jnp.where(flat == sel, _NEG, s)
    bs_ref[...] = bs_v
    nw_ref[...] = nw_v
    pi_ref[...] = pi_v
    np_ref[...] = np_v


def kernel(word_prob, prob, stops, word_length):
    BB, V = word_prob.shape
    nb = BB // _BEAM
    p3 = prob.reshape(nb, 1, _BEAM).astype(jnp.float32)
    st3 = stops.reshape(nb, 1, _BEAM).astype(jnp.int32)
    wl3 = word_length.reshape(nb, 1, _BEAM).astype(jnp.int32)

    small = pl.BlockSpec((1, 1, _BEAM), lambda g: (g, 0, 0))
    out_shape = [
        jax.ShapeDtypeStruct((nb, 1, _BEAM), jnp.float32),
        jax.ShapeDtypeStruct((nb, 1, _BEAM), jnp.int32),
        jax.ShapeDtypeStruct((nb, 1, _BEAM), jnp.int32),
        jax.ShapeDtypeStruct((nb, 1, _BEAM), jnp.float32),
    ]
    bs, nw, pi, npb = pl.pallas_call(
        _group_kernel,
        grid=(nb,),
        in_specs=[
            pl.BlockSpec((_BEAM, V), lambda g: (g, 0)),
            small, small, small,
        ],
        out_specs=[small, small, small, small],
        out_shape=out_shape,
    )(word_prob, p3, st3, wl3)
    return (bs.reshape(nb, _BEAM), nw.reshape(-1), pi.reshape(-1),
            npb.reshape(-1))


# speedup vs baseline: 1.3613x; 1.0293x over previous
import jax
import jax.numpy as jnp
from jax.experimental import pallas as pl

_BEAM = 8
_PAD = 0
_EOS = 2
_NEG = -3.0e38
_IMAX = 2147483647
_FLOOR = -46.0517018598809136


def _group_kernel(wp_ref, p_ref, st_ref, wl_ref, bs_ref, nw_ref, pi_ref, np_ref):
    g = pl.program_id(0)
    V = wp_ref.shape[1]
    w = wp_ref[...]
    p = p_ref[0, 0, :].reshape(_BEAM, 1)
    st = st_ref[0, 0, :].reshape(_BEAM, 1)
    wl = wl_ref[0, 0, :].reshape(_BEAM, 1)
    live = st == 0
    wlf = wl.astype(jnp.float32)
    lp0 = (wlf + 5.0) * (1.0 / 6.0)
    lp1 = (wlf + 6.0) * (1.0 / 6.0)

    col = jax.lax.broadcasted_iota(jnp.int32, (_BEAM, V), 1)

    valid = (col != _PAD) & (col != _EOS) & live
    wm = jnp.where(valid, w, -1.0)
    lane8 = jax.lax.broadcasted_iota(jnp.int32, (_BEAM, _BEAM), 1)
    cand_w = jnp.zeros((_BEAM, _BEAM), jnp.float32)
    cand_c = jnp.zeros((_BEAM, _BEAM), jnp.int32)
    for k in range(_BEAM):
        m = jnp.max(wm, axis=1, keepdims=True)
        selc = jnp.min(jnp.where(wm == m, col, _IMAX), axis=1,
                       keepdims=True)
        cand_w = jnp.where(lane8 == k, m, cand_w)
        cand_c = jnp.where(lane8 == k, selc, cand_c)
        if k < _BEAM - 1:
            wm = jnp.where(col == selc, -2.0, wm)

    row8 = jax.lax.broadcasted_iota(jnp.int32, (_BEAM, _BEAM), 0)
    cand_wp = p * cand_w
    cand_s = jnp.log(jnp.clip(cand_wp, 1e-20, 1.0)) / lp1
    cand_s = jnp.where(cand_w >= 0.0, cand_s, _NEG)
    cand_f = row8 * V + cand_c

    w0 = w[:, _PAD:_PAD + 1]
    w2 = w[:, _EOS:_EOS + 1]
    wp_pad = jnp.where(live, p * w0, p)
    s_pad = jnp.log(jnp.clip(wp_pad, 1e-20, 1.0)) / lp0
    wp_eos = jnp.where(live, p * w2, 0.0)
    s_eos = jnp.log(jnp.clip(wp_eos, 1e-20, 1.0)) / lp0
    s_floor = _FLOOR / lp0

    spec_s = jnp.where(lane8 == 0, s_pad,
                       jnp.where(lane8 == 1, s_eos,
                                 jnp.where(live, _NEG, s_floor)))
    spec_wp = jnp.where(lane8 == 0, wp_pad,
                        jnp.where(lane8 == 1, wp_eos, 0.0))
    spec_c = jnp.where(lane8 == 0, _PAD,
                       jnp.where(lane8 == 1, _EOS,
                                 jnp.where(lane8 == 2, 1, lane8)))
    spec_f = row8 * V + spec_c

    sc = jnp.concatenate([cand_s, spec_s], axis=1)
    fc = jnp.concatenate([cand_f, spec_f], axis=1)
    wc = jnp.concatenate([cand_wp, spec_wp], axis=1)

    lane = jax.lax.broadcasted_iota(jnp.int32, (1, 1, _BEAM), 2)
    bs_v = jnp.zeros((1, 1, _BEAM), jnp.float32)
    nw_v = jnp.zeros((1, 1, _BEAM), jnp.int32)
    pi_v = jnp.zeros((1, 1, _BEAM), jnp.int32)
    np_v = jnp.zeros((1, 1, _BEAM), jnp.float32)
    for k in range(_BEAM):
        m = jnp.max(sc)
        sel = jnp.min(jnp.where(sc == m, fc, _IMAX))
        wsel = jnp.max(jnp.where(fc == sel, wc, -1.0))
        bs_v = jnp.where(lane == k, m, bs_v)
        nw_v = jnp.where(lane == k, sel % V, nw_v)
        pi_v = jnp.where(lane == k, g * _BEAM + sel // V, pi_v)
        np_v = jnp.where(lane == k, wsel, np_v)
        if k < _BEAM - 1:
            sc = jnp.where(fc == sel, _NEG, sc)
    bs_ref[...] = bs_v
    nw_ref[...] = nw_v
    pi_ref[...] = pi_v
    np_ref[...] = np_v


def kernel(word_prob, prob, stops, word_length):
    BB, V = word_prob.shape
    nb = BB // _BEAM
    p3 = prob.reshape(nb, 1, _BEAM).astype(jnp.float32)
    st3 = stops.reshape(nb, 1, _BEAM).astype(jnp.int32)
    wl3 = word_length.reshape(nb, 1, _BEAM).astype(jnp.int32)

    small = pl.BlockSpec((1, 1, _BEAM), lambda g: (g, 0, 0))
    out_shape = [
        jax.ShapeDtypeStruct((nb, 1, _BEAM), jnp.float32),
        jax.ShapeDtypeStruct((nb, 1, _BEAM), jnp.int32),
        jax.ShapeDtypeStruct((nb, 1, _BEAM), jnp.int32),
        jax.ShapeDtypeStruct((nb, 1, _BEAM), jnp.float32),
    ]
    bs, nw, pi, npb = pl.pallas_call(
        _group_kernel,
        grid=(nb,),
        in_specs=[
            pl.BlockSpec((_BEAM, V), lambda g: (g, 0)),
            small, small, small,
        ],
        out_specs=[small, small, small, small],
        out_shape=out_shape,
    )(word_prob, p3, st3, wl3)
    return (bs.reshape(nb, _BEAM), nw.reshape(-1), pi.reshape(-1),
            npb.reshape(-1))


# speedup vs baseline: 1.6107x; 1.1832x over previous
import jax
import jax.numpy as jnp
from jax.experimental import pallas as pl

_BEAM = 8
_PAD = 0
_EOS = 2
_NEG = -3.0e38
_IMAX = 2147483647
_FLOOR = -46.0517018598809136

_SUB = 160
_SUBW = 625
_KEEP = 10


def _group_kernel(wp_ref, p_ref, st_ref, wl_ref, bs_ref, nw_ref, pi_ref, np_ref):
    g = pl.program_id(0)
    blk = wp_ref[...]
    p = p_ref[0, 0, :].reshape(_BEAM, 1)
    st = st_ref[0, 0, :].reshape(_BEAM, 1)
    wl = wl_ref[0, 0, :].reshape(_BEAM, 1)
    live = st == 0
    wlf = wl.astype(jnp.float32)
    lp0 = (wlf + 5.0) * (1.0 / 6.0)
    lp1 = (wlf + 6.0) * (1.0 / 6.0)

    M8 = jnp.max(blk, axis=1, keepdims=True).reshape(_BEAM, _SUB)
    c160 = jax.lax.broadcasted_iota(jnp.int32, (_BEAM, _SUB), 1)
    r8 = jax.lax.broadcasted_iota(jnp.int32, (_BEAM, 1), 0)
    c1280b = jax.lax.broadcasted_iota(jnp.int32, (_BEAM, _BEAM * _SUB), 1)
    j625 = jax.lax.broadcasted_iota(jnp.int32, (_BEAM, _SUBW), 1)
    ohs = []
    gs = []
    mwork = M8
    for k in range(_KEEP):
        mk = jnp.max(mwork, axis=1, keepdims=True)
        sk = jnp.min(jnp.where(mwork == mk, c160, _IMAX), axis=1,
                     keepdims=True)
        ohs.append((c1280b == (r8 * _SUB + sk)).astype(jnp.float32))
        gs.append(sk * _SUBW + j625)
        if k < _KEEP - 1:
            mwork = jnp.where(c160 == sk, -1.0, mwork)

    oh = jnp.concatenate(ohs, axis=0)
    c80 = jax.lax.dot_general(oh, blk, (((1,), (0,)), ((), ())),
                              preferred_element_type=jnp.float32,
                              precision=jax.lax.Precision.HIGHEST)
    cmp = jnp.concatenate(
        [c80[k * _BEAM:(k + 1) * _BEAM, :] for k in range(_KEEP)], axis=1)
    gcol = jnp.concatenate(gs, axis=1)

    valid = (gcol != _PAD) & (gcol != _EOS) & live
    wm = jnp.where(valid, cmp, -1.0)
    lane8 = jax.lax.broadcasted_iota(jnp.int32, (_BEAM, _BEAM), 1)
    cand_w = jnp.zeros((_BEAM, _BEAM), jnp.float32)
    cand_c = jnp.zeros((_BEAM, _BEAM), jnp.int32)
    for k in range(_BEAM):
        m = jnp.max(wm, axis=1, keepdims=True)
        selc = jnp.min(jnp.where(wm == m, gcol, _IMAX), axis=1,
                       keepdims=True)
        cand_w = jnp.where(lane8 == k, m, cand_w)
        cand_c = jnp.where(lane8 == k, selc, cand_c)
        if k < _BEAM - 1:
            wm = jnp.where(gcol == selc, -2.0, wm)

    V = _SUB * _SUBW
    row8 = jax.lax.broadcasted_iota(jnp.int32, (_BEAM, _BEAM), 0)
    cand_wp = p * cand_w
    cand_s = jnp.log(jnp.clip(cand_wp, 1e-20, 1.0)) / lp1
    cand_s = jnp.where(cand_w >= 0.0, cand_s, _NEG)
    cand_f = row8 * V + cand_c

    r8 = jax.lax.broadcasted_iota(jnp.int32, (_BEAM, 1), 0)
    c1280b = jax.lax.broadcasted_iota(jnp.int32, (_BEAM, _BEAM * _SUB), 1)
    oh0 = (c1280b == r8 * _SUB).astype(jnp.float32)
    row0 = jax.lax.dot_general(oh0, blk, (((1,), (0,)), ((), ())),
                               preferred_element_type=jnp.float32,
                               precision=jax.lax.Precision.HIGHEST)
    w0r = row0[:, _PAD:_PAD + 1]
    w2r = row0[:, _EOS:_EOS + 1]

    wp_pad = jnp.where(live, p * w0r, p)
    s_pad = jnp.log(jnp.clip(wp_pad, 1e-20, 1.0)) / lp0
    wp_eos = jnp.where(live, p * w2r, 0.0)
    s_eos = jnp.log(jnp.clip(wp_eos, 1e-20, 1.0)) / lp0
    s_floor = _FLOOR / lp0

    spec_s = jnp.where(lane8 == 0, s_pad,
                       jnp.where(lane8 == 1, s_eos,
                                 jnp.where(live, _NEG, s_floor)))
    spec_wp = jnp.where(lane8 == 0, wp_pad,
                        jnp.where(lane8 == 1, wp_eos, 0.0))
    spec_c = jnp.where(lane8 == 0, _PAD,
                       jnp.where(lane8 == 1, _EOS,
                                 jnp.where(lane8 == 2, 1, lane8)))
    spec_f = row8 * V + spec_c

    sc = jnp.concatenate([cand_s, spec_s], axis=1)
    fc = jnp.concatenate([cand_f, spec_f], axis=1)
    wc = jnp.concatenate([cand_wp, spec_wp], axis=1)

    lane = jax.lax.broadcasted_iota(jnp.int32, (1, 1, _BEAM), 2)
    bs_v = jnp.zeros((1, 1, _BEAM), jnp.float32)
    nw_v = jnp.zeros((1, 1, _BEAM), jnp.int32)
    pi_v = jnp.zeros((1, 1, _BEAM), jnp.int32)
    np_v = jnp.zeros((1, 1, _BEAM), jnp.float32)
    for k in range(_BEAM):
        m = jnp.max(sc)
        sel = jnp.min(jnp.where(sc == m, fc, _IMAX))
        wsel = jnp.max(jnp.where(fc == sel, wc, -1.0))
        bs_v = jnp.where(lane == k, m, bs_v)
        nw_v = jnp.where(lane == k, sel % V, nw_v)
        pi_v = jnp.where(lane == k, g * _BEAM + sel // V, pi_v)
        np_v = jnp.where(lane == k, wsel, np_v)
        if k < _BEAM - 1:
            sc = jnp.where(fc == sel, _NEG, sc)
    bs_ref[...] = bs_v
    nw_ref[...] = nw_v
    pi_ref[...] = pi_v
    np_ref[...] = np_v


def kernel(word_prob, prob, stops, word_length):
    BB, V = word_prob.shape
    nb = BB // _BEAM
    wsub = word_prob.reshape(BB * _SUB, _SUBW)
    p3 = prob.reshape(nb, 1, _BEAM).astype(jnp.float32)
    st3 = stops.reshape(nb, 1, _BEAM).astype(jnp.int32)
    wl3 = word_length.reshape(nb, 1, _BEAM).astype(jnp.int32)

    small = pl.BlockSpec((1, 1, _BEAM), lambda g: (g, 0, 0))
    out_shape = [
        jax.ShapeDtypeStruct((nb, 1, _BEAM), jnp.float32),
        jax.ShapeDtypeStruct((nb, 1, _BEAM), jnp.int32),
        jax.ShapeDtypeStruct((nb, 1, _BEAM), jnp.int32),
        jax.ShapeDtypeStruct((nb, 1, _BEAM), jnp.float32),
    ]
    bs, nw, pi, npb = pl.pallas_call(
        _group_kernel,
        grid=(nb,),
        in_specs=[
            pl.BlockSpec((_BEAM * _SUB, _SUBW), lambda g: (g, 0)),
            small, small, small,
        ],
        out_specs=[small, small, small, small],
        out_shape=out_shape,
    )(wsub, p3, st3, wl3)
    return (bs.reshape(nb, _BEAM), nw.reshape(-1), pi.reshape(-1),
            npb.reshape(-1))


# speedup vs baseline: 1.8554x; 1.1520x over previous
import jax
import jax.numpy as jnp
from jax.experimental import pallas as pl
from jax.experimental.pallas import tpu as pltpu

_BEAM = 8
_PAD = 0
_EOS = 2
_NEG = -3.0e38
_IMAX = 2147483647
_FLOOR = -46.0517018598809136

_SUB = 160
_SUBW = 625
_KEEP = 10


def _group_kernel(wp_ref, p_ref, st_ref, wl_ref, bs_ref, nw_ref, pi_ref,
                  np_ref, scr_ref):
    g = pl.program_id(0)
    blk = wp_ref[...]
    p = p_ref[0, 0, :].reshape(_BEAM, 1)
    st = st_ref[0, 0, :].reshape(_BEAM, 1)
    wl = wl_ref[0, 0, :].reshape(_BEAM, 1)
    live = st == 0
    wlf = wl.astype(jnp.float32)
    lp0 = (wlf + 5.0) * (1.0 / 6.0)
    lp1 = (wlf + 6.0) * (1.0 / 6.0)

    M8 = jnp.max(blk, axis=1, keepdims=True).reshape(_BEAM, _SUB)
    c160 = jax.lax.broadcasted_iota(jnp.int32, (_BEAM, _SUB), 1)
    r8 = jax.lax.broadcasted_iota(jnp.int32, (_BEAM, 1), 0)
    j625 = jax.lax.broadcasted_iota(jnp.int32, (_BEAM, _SUBW), 1)
    gs = []
    mwork = M8
    for k in range(_KEEP):
        mk = jnp.max(mwork, axis=1, keepdims=True)
        sk = jnp.min(jnp.where(mwork == mk, c160, _IMAX), axis=1,
                     keepdims=True)
        gs.append(sk * _SUBW + j625)
        for r in range(_BEAM):
            t = jnp.max(jnp.where(r8 == r, sk, 0))
            q = k * _BEAM + r
            scr_ref[q:q + 1, :] = wp_ref[pl.ds(r * _SUB + t, 1), :]
        if k < _KEEP - 1:
            mwork = jnp.where(c160 == sk, -1.0, mwork)

    c80 = scr_ref[...]
    cmp = jnp.concatenate(
        [c80[k * _BEAM:(k + 1) * _BEAM, :] for k in range(_KEEP)], axis=1)
    gcol = jnp.concatenate(gs, axis=1)

    valid = (gcol != _PAD) & (gcol != _EOS) & live
    wm = jnp.where(valid, cmp, -1.0)
    lane8 = jax.lax.broadcasted_iota(jnp.int32, (_BEAM, _BEAM), 1)
    cand_w = jnp.zeros((_BEAM, _BEAM), jnp.float32)
    cand_c = jnp.zeros((_BEAM, _BEAM), jnp.int32)
    for k in range(_BEAM):
        m = jnp.max(wm, axis=1, keepdims=True)
        selc = jnp.min(jnp.where(wm == m, gcol, _IMAX), axis=1,
                       keepdims=True)
        cand_w = jnp.where(lane8 == k, m, cand_w)
        cand_c = jnp.where(lane8 == k, selc, cand_c)
        if k < _BEAM - 1:
            wm = jnp.where(gcol == selc, -2.0, wm)

    V = _SUB * _SUBW
    row8 = jax.lax.broadcasted_iota(jnp.int32, (_BEAM, _BEAM), 0)
    cand_wp = p * cand_w
    cand_s = jnp.log(jnp.clip(cand_wp, 1e-20, 1.0)) / lp1
    cand_s = jnp.where(cand_w >= 0.0, cand_s, _NEG)
    cand_f = row8 * V + cand_c

    w0r = jnp.concatenate(
        [blk[r * _SUB:r * _SUB + 1, _PAD:_PAD + 1] for r in range(_BEAM)],
        axis=0)
    w2r = jnp.concatenate(
        [blk[r * _SUB:r * _SUB + 1, _EOS:_EOS + 1] for r in range(_BEAM)],
        axis=0)

    wp_pad = jnp.where(live, p * w0r, p)
    s_pad = jnp.log(jnp.clip(wp_pad, 1e-20, 1.0)) / lp0
    wp_eos = jnp.where(live, p * w2r, 0.0)
    s_eos = jnp.log(jnp.clip(wp_eos, 1e-20, 1.0)) / lp0
    s_floor = _FLOOR / lp0

    spec_s = jnp.where(lane8 == 0, s_pad,
                       jnp.where(lane8 == 1, s_eos,
                                 jnp.where(live, _NEG, s_floor)))
    spec_wp = jnp.where(lane8 == 0, wp_pad,
                        jnp.where(lane8 == 1, wp_eos, 0.0))
    spec_c = jnp.where(lane8 == 0, _PAD,
                       jnp.where(lane8 == 1, _EOS,
                                 jnp.where(lane8 == 2, 1, lane8)))
    spec_f = row8 * V + spec_c

    sc = jnp.concatenate([cand_s, spec_s], axis=1)
    fc = jnp.concatenate([cand_f, spec_f], axis=1)
    wc = jnp.concatenate([cand_wp, spec_wp], axis=1)

    lane = jax.lax.broadcasted_iota(jnp.int32, (1, 1, _BEAM), 2)
    bs_v = jnp.zeros((1, 1, _BEAM), jnp.float32)
    nw_v = jnp.zeros((1, 1, _BEAM), jnp.int32)
    pi_v = jnp.zeros((1, 1, _BEAM), jnp.int32)
    np_v = jnp.zeros((1, 1, _BEAM), jnp.float32)
    for k in range(_BEAM):
        m = jnp.max(sc)
        sel = jnp.min(jnp.where(sc == m, fc, _IMAX))
        wsel = jnp.max(jnp.where(fc == sel, wc, -1.0))
        bs_v = jnp.where(lane == k, m, bs_v)
        nw_v = jnp.where(lane == k, sel % V, nw_v)
        pi_v = jnp.where(lane == k, g * _BEAM + sel // V, pi_v)
        np_v = jnp.where(lane == k, wsel, np_v)
        if k < _BEAM - 1:
            sc = jnp.where(fc == sel, _NEG, sc)
    bs_ref[...] = bs_v
    nw_ref[...] = nw_v
    pi_ref[...] = pi_v
    np_ref[...] = np_v


def kernel(word_prob, prob, stops, word_length):
    BB, V = word_prob.shape
    nb = BB // _BEAM
    wsub = word_prob.reshape(BB * _SUB, _SUBW)
    p3 = prob.reshape(nb, 1, _BEAM).astype(jnp.float32)
    st3 = stops.reshape(nb, 1, _BEAM).astype(jnp.int32)
    wl3 = word_length.reshape(nb, 1, _BEAM).astype(jnp.int32)

    small = pl.BlockSpec((1, 1, _BEAM), lambda g: (g, 0, 0))
    out_shape = [
        jax.ShapeDtypeStruct((nb, 1, _BEAM), jnp.float32),
        jax.ShapeDtypeStruct((nb, 1, _BEAM), jnp.int32),
        jax.ShapeDtypeStruct((nb, 1, _BEAM), jnp.int32),
        jax.ShapeDtypeStruct((nb, 1, _BEAM), jnp.float32),
    ]
    bs, nw, pi, npb = pl.pallas_call(
        _group_kernel,
        grid=(nb,),
        in_specs=[
            pl.BlockSpec((_BEAM * _SUB, _SUBW), lambda g: (g, 0)),
            small, small, small,
        ],
        out_specs=[small, small, small, small],
        out_shape=out_shape,
        scratch_shapes=[pltpu.VMEM((_BEAM * _KEEP, _SUBW), jnp.float32)],
    )(wsub, p3, st3, wl3)
    return (bs.reshape(nb, _BEAM), nw.reshape(-1), pi.reshape(-1),
            npb.reshape(-1))
